# Initial kernel scaffold; baseline (speedup 1.0000x reference)
import jax
import jax.numpy as jnp
from jax.experimental import pallas as pl


def kernel(data, indices, index):
    raise NotImplementedError("write your pallas kernel here")


# speedup vs baseline: 1.1295x; 1.1295x over previous
import functools

import jax
import jax.numpy as jnp
from jax import lax
from jax.experimental import pallas as pl
from jax.experimental.pallas import tpu as pltpu
from jax.experimental.pallas import tpu_sc as plsc

BATCH = 4096
N_DIMS = 128
NUM_CORES = 2
NUM_SUBCORES = 16
NUM_WORKERS = NUM_CORES * NUM_SUBCORES
ROWS_PER_WORKER = BATCH // NUM_WORKERS


@functools.partial(jax.jit, static_argnames=())
def _gather_sc(data, indices, positions):
    mesh = plsc.VectorSubcoreMesh(core_axis_name="c", subcore_axis_name="s")

    @functools.partial(
        pl.kernel,
        mesh=mesh,
        out_type=jax.ShapeDtypeStruct((BATCH, N_DIMS), jnp.float32),
        scratch_types=[
            pltpu.VMEM((ROWS_PER_WORKER,), jnp.int32),
            pltpu.VMEM((ROWS_PER_WORKER,), jnp.int32),
            pltpu.VMEM((ROWS_PER_WORKER, N_DIMS), jnp.float32),
            pltpu.SemaphoreType.DMA,
        ],
    )
    def body(data_hbm, idx_hbm, pos_hbm, out_hbm, pos_v, val_v, rows_v, sem):
        wid = lax.axis_index("s") * NUM_CORES + lax.axis_index("c")
        base = wid * ROWS_PER_WORKER
        pltpu.sync_copy(pos_hbm.at[pl.ds(base, ROWS_PER_WORKER)], pos_v)
        pltpu.async_copy(idx_hbm.at[pos_v], val_v, sem).wait()
        pltpu.async_copy(data_hbm.at[val_v], rows_v, sem).wait()
        pltpu.sync_copy(rows_v, out_hbm.at[pl.ds(base, ROWS_PER_WORKER)])

    return body(data, indices, positions)


def kernel(data, indices, index):
    n = indices.shape[0]
    index = jnp.asarray(index, jnp.int32)
    break_condition = index >= n
    new_index = index + BATCH
    start = jnp.clip(index, 0, n - BATCH)
    positions = start + jnp.arange(BATCH, dtype=jnp.int32)
    batch = _gather_sc(data, indices, positions)
    return (batch, new_index, break_condition)
